# Initial kernel scaffold; baseline (speedup 1.0000x reference)
import jax
import jax.numpy as jnp
from jax.experimental import pallas as pl


def kernel(x, edge_index, current_node, W_gnn, b_gnn, W_fc, b_fc):
    raise NotImplementedError("write your pallas kernel here")


# speedup vs baseline: 4.6584x; 4.6584x over previous
import functools

import jax
import jax.numpy as jnp
from jax import lax
from jax.experimental import pallas as pl
from jax.experimental.pallas import tpu as pltpu
from jax.experimental.pallas import tpu_sc as plsc

N_NODES = 10000
N_EDGES = 320000
D_FEAT = 128
HIDDEN_DIM = 128
MAX_COLORS = 16
BATCH_NODES = 1024

NC = 2
NS = 16
L = 16

CHUNK = 128
E_T = N_EDGES // NS
N_SEG = 5
E_SEG = E_T // N_SEG
N_GRP = E_SEG // L
C_CAP = -(-E_SEG // CHUNK)
N_SEL = 10112
RH = 5120
RS = RH + CHUNK
R_PAD = NC * RH
RPT = RS // NS

_mesh = plsc.VectorSubcoreMesh(
    core_axis_name="c", subcore_axis_name="s", num_cores=NC, num_subcores=NS)


@functools.partial(
    pl.kernel,
    out_type=[
        jax.ShapeDtypeStruct((NC, RS, D_FEAT), jnp.float32),
        jax.ShapeDtypeStruct((NC * RS,), jnp.float32),
    ],
    mesh=_mesh,
    compiler_params=pltpu.CompilerParams(needs_layout_passes=False),
    scratch_types=[
        pltpu.VMEM((N_SEL,), jnp.int32),
        pltpu.VMEM((BATCH_NODES,), jnp.int32),
        pltpu.VMEM((E_SEG + 96,), jnp.int32),
        pltpu.VMEM((E_SEG + 96,), jnp.int32),
        pltpu.VMEM((C_CAP, CHUNK), jnp.int32),
        pltpu.VMEM((C_CAP, CHUNK), jnp.int32),
        pltpu.VMEM((CHUNK, D_FEAT), jnp.float32),
        pltpu.VMEM((CHUNK,), jnp.float32),
        pltpu.VMEM((RPT + 56,), jnp.float32),
        pltpu.VMEM_SHARED((RS, D_FEAT), jnp.float32),
        pltpu.VMEM_SHARED((RS,), jnp.float32),
        pltpu.SemaphoreType.DMA,
    ],
)
def _sc_aggregate(src_hbm, dst_hbm, cn_hbm, x_hbm, acc_out, deg_out,
                  sel_v, cn_v, src_e, dst_e, comp_src, comp_dst,
                  rowbuf, onesv, zdeg, acc_sh, deg_sh, sem):
    cid = lax.axis_index("c")
    sid = lax.axis_index("s")

    zero16 = jnp.zeros((L,), jnp.float32)
    ones16 = jnp.ones((L,), jnp.float32)
    iota16 = lax.iota(jnp.int32, L)

    def _fill(i, c):
        for j in range(D_FEAT // L):
            rowbuf[i, pl.ds(j * L, L)] = zero16
        return c

    lax.fori_loop(0, CHUNK, _fill, 0)
    for j in range(CHUNK // L):
        onesv[pl.ds(j * L, L)] = ones16
    for j in range((RPT + 56) // L):
        zdeg[pl.ds(j * L, L)] = zero16

    row0 = sid * RPT
    for k0, sz in ((0, CHUNK), (CHUNK, CHUNK), (2 * CHUNK, RPT - 2 * CHUNK)):
        pltpu.sync_copy(rowbuf.at[pl.ds(0, sz)],
                        acc_sh.at[pl.ds(row0 + k0, sz)])
    pltpu.sync_copy(zdeg.at[pl.ds(0, RPT)], deg_sh.at[pl.ds(row0, RPT)])

    def _selz(i, c):
        sel_v[pl.ds(i * L, L)] = jnp.zeros((L,), jnp.int32)
        return c

    lax.fori_loop(0, N_SEL // L, _selz, 0)
    pltpu.sync_copy(cn_hbm, cn_v)

    def _sels(i, c):
        plsc.store_scatter(sel_v, [cn_v[pl.ds(i * L, L)]],
                           jnp.ones((L,), jnp.int32),
                           mask=jnp.full((L,), True))
        return c

    lax.fori_loop(0, BATCH_NODES // L, _sels, 0)

    plsc.subcore_barrier()

    dump16 = jnp.full((L,), RH, jnp.int32)

    def _segment(seg, carry):
        e0 = sid * E_T + seg * E_SEG
        pltpu.sync_copy(src_hbm.at[pl.ds(e0, E_SEG)], src_e.at[pl.ds(0, E_SEG)])
        pltpu.sync_copy(dst_hbm.at[pl.ds(e0, E_SEG)], dst_e.at[pl.ds(0, E_SEG)])

        def _grp(i, cur):
            s16 = src_e[pl.ds(i * L, L)]
            d16 = dst_e[pl.ds(i * L, L)]
            selv = plsc.load_gather(sel_v, [d16])
            dloc = d16 - cid * RH
            m = (selv > 0) & (dloc >= 0) & (dloc < RH)
            mi = m.astype(jnp.int32)
            inc = plsc.cumsum(mi)
            pos = cur + inc - 1
            plsc.store_scatter(comp_src,
                               [pos >> 7, pos & (CHUNK - 1)], s16, mask=m)
            plsc.store_scatter(comp_dst,
                               [pos >> 7, pos & (CHUNK - 1)], dloc, mask=m)
            return cur + jnp.max(inc)

        cnt = lax.fori_loop(0, N_GRP, _grp, jnp.int32(0))

        cend = (cnt + CHUNK - 1) & ~(CHUNK - 1)
        for g in range(CHUNK // L):
            p = cnt + g * L + iota16
            mm = p < cend
            plsc.store_scatter(comp_src, [p >> 7, p & (CHUNK - 1)],
                               jnp.zeros((L,), jnp.int32), mask=mm)
            plsc.store_scatter(comp_dst, [p >> 7, p & (CHUNK - 1)],
                               dump16, mask=mm)

        def _chunk(j, c):
            pltpu.async_copy(x_hbm.at[comp_src.at[j]], rowbuf, sem).wait()
            pltpu.sync_copy(rowbuf, acc_sh.at[comp_dst.at[j]], add=True)
            pltpu.sync_copy(onesv, deg_sh.at[comp_dst.at[j]], add=True)
            return c

        lax.fori_loop(0, cend >> 7, _chunk, 0)
        return carry

    lax.fori_loop(0, N_SEG, _segment, 0)
    plsc.subcore_barrier()

    pltpu.sync_copy(acc_sh.at[pl.ds(row0, RPT)],
                    acc_out.at[cid, pl.ds(row0, RPT)])
    pltpu.sync_copy(deg_sh.at[pl.ds(row0, RPT)], zdeg.at[pl.ds(0, RPT)])
    pltpu.sync_copy(zdeg.at[pl.ds(0, RPT)],
                    deg_out.at[pl.ds(cid * RS + row0, RPT)])


_RB = 512
_NB = RH // _RB


def _tc_dense_body(acc_ref, deg_ref, wg_ref, bg_ref, wf_ref, bf_ref, out_ref):
    a = acc_ref[0]
    d = jnp.maximum(deg_ref[0], 1.0)
    e = jnp.maximum(
        jnp.dot(a / d, wg_ref[...], preferred_element_type=jnp.float32)
        + bg_ref[...][None, :], 0.0)
    f = (jnp.dot(e, wf_ref[...], preferred_element_type=jnp.float32)
         + bf_ref[...][None, :])
    out_ref[...] = jnp.concatenate(
        [f, jnp.zeros((_RB, D_FEAT - MAX_COLORS), jnp.float32)], axis=1)


def _tc_dense(acc, deg, W_gnn, b_gnn, W_fc, b_fc):
    return pl.pallas_call(
        _tc_dense_body,
        grid=(NC * _NB,),
        in_specs=[
            pl.BlockSpec((1, _RB, D_FEAT), lambda i: (i // _NB, i % _NB, 0)),
            pl.BlockSpec((1, _RB, 1), lambda i: (i // _NB, i % _NB, 0)),
            pl.BlockSpec((D_FEAT, HIDDEN_DIM), lambda i: (0, 0)),
            pl.BlockSpec((HIDDEN_DIM,), lambda i: (0,)),
            pl.BlockSpec((HIDDEN_DIM, MAX_COLORS), lambda i: (0, 0)),
            pl.BlockSpec((MAX_COLORS,), lambda i: (0,)),
        ],
        out_specs=pl.BlockSpec((_RB, D_FEAT), lambda i: (i, 0)),
        out_shape=jax.ShapeDtypeStruct((R_PAD, D_FEAT), jnp.float32),
    )(acc, deg, W_gnn, b_gnn, W_fc, b_fc)


_B_W = BATCH_NODES // (NC * NS)


@functools.partial(
    pl.kernel,
    out_type=jax.ShapeDtypeStruct((BATCH_NODES, D_FEAT), jnp.float32),
    mesh=_mesh,
    compiler_params=pltpu.CompilerParams(needs_layout_passes=False),
    scratch_types=[
        pltpu.VMEM((_B_W,), jnp.int32),
        pltpu.VMEM((_B_W, D_FEAT), jnp.float32),
        pltpu.SemaphoreType.DMA,
    ],
)
def _sc_select(f_hbm, cn_hbm, out_hbm, idx_v, rows_v, sem):
    wid = lax.axis_index("c") * NS + lax.axis_index("s")
    base = wid * _B_W
    pltpu.sync_copy(cn_hbm.at[pl.ds(base, _B_W)], idx_v)
    pltpu.async_copy(f_hbm.at[idx_v], rows_v, sem).wait()
    pltpu.sync_copy(rows_v, out_hbm.at[pl.ds(base, _B_W)])


def kernel(x, edge_index, current_node, W_gnn, b_gnn, W_fc, b_fc):
    acc, deg = _sc_aggregate(edge_index[0], edge_index[1], current_node, x)
    logits = _tc_dense(acc, deg.reshape(NC, RS, 1), W_gnn, b_gnn, W_fc, b_fc)
    return _sc_select(logits, current_node)[:, :MAX_COLORS]
